# baseline (speedup 1.0000x reference)
import functools

import jax
import jax.numpy as jnp
from jax import lax
from jax.experimental import pallas as pl
from jax.experimental.pallas import tpu as pltpu
from jax.experimental.pallas import tpu_sc as plsc

DIM = 32
WPR = DIM // 2
BATCH = 16384
NROWS = 1000000
L = 16
NC = 2
NS = 16
NW = NC * NS
BPW = BATCH // NW
CHUNK = 128
NCH = BPW // CHUNK
RPL = 8
NLINES = NROWS // RPL

_mesh = plsc.VectorSubcoreMesh(core_axis_name="c", subcore_axis_name="s")


@functools.partial(
    pl.kernel,
    out_type=jax.ShapeDtypeStruct((BATCH,), jnp.float32),
    mesh=_mesh,
    scratch_types=[
        pltpu.VMEM((3 * BPW,), jnp.int32),
        pltpu.VMEM((3 * BPW,), jnp.int32),
        pltpu.VMEM((3 * BPW,), jnp.int32),
        pltpu.VMEM((3, CHUNK, 128), jnp.int32),
        pltpu.VMEM((3, CHUNK, 128), jnp.int32),
        pltpu.VMEM((BPW,), jnp.float32),
        pltpu.SemaphoreType.DMA,
        pltpu.SemaphoreType.DMA,
    ],
    compiler_params=pltpu.CompilerParams(needs_layout_passes=False),
)
def _transe_sc(ent_l, rel_l, idx, out,
               idx_v, line_v, offs_v, st0, st1, out_v, sem0, sem1):
    wid = lax.axis_index("s") * NC + lax.axis_index("c")
    base = wid * BPW

    pltpu.sync_copy(idx.at[pl.ds(wid * (3 * BPW), 3 * BPW)], idx_v)

    def mkline(i, carry):
        v = idx_v[pl.ds(i * L, L)]
        line_v[pl.ds(i * L, L)] = lax.shift_right_logical(v, 3)
        offs_v[pl.ds(i * L, L)] = (v & (RPL - 1)) * WPR
        return carry

    lax.fori_loop(0, (3 * BPW) // L, mkline, 0)

    tables = (ent_l, rel_l, ent_l)
    stages = (st0, st1)
    sems = (sem0, sem1)

    def fire(k):
        st = stages[k % 2]
        sem = sems[k % 2]
        return [
            pltpu.async_copy(
                tables[t].at[line_v.at[pl.ds(t * BPW + k * CHUNK, CHUNK)]],
                st.at[t], sem)
            for t in range(3)
        ]

    def word(st, tsel, lanes, offs, t, j):
        w = plsc.load_gather(st, [tsel[t], lanes, offs[t] + j])
        bf = plsc.bitcast(w, jnp.bfloat16)
        return plsc.unpack(bf, format=plsc.PackFormat.INTERLEAVED)

    pending = {0: fire(0)}
    for k in range(NCH):
        if k + 1 < NCH:
            pending[k + 1] = fire(k + 1)
        for c in pending.pop(k):
            c.wait()
        st = stages[k % 2]

        def group(g, carry, k=k, st=st):
            lanes = g * L + lax.iota(jnp.int32, L)
            tsel = [jnp.full((L,), t, jnp.int32) for t in range(3)]
            offs = []
            for t in range(3):
                offs.append(offs_v[pl.ds(t * BPW + k * CHUNK + g * L, L)])
            n1 = jnp.zeros((L,), jnp.float32)
            nr = jnp.zeros((L,), jnp.float32)
            n2 = jnp.zeros((L,), jnp.float32)
            for j in range(WPR):
                a, b = word(st, tsel, lanes, offs, 0, j)
                n1 = n1 + (jnp.abs(a) + jnp.abs(b))
                a, b = word(st, tsel, lanes, offs, 1, j)
                nr = nr + (jnp.abs(a) + jnp.abs(b))
                a, b = word(st, tsel, lanes, offs, 2, j)
                n2 = n2 + (jnp.abs(a) + jnp.abs(b))
            s1 = 1.0 / n1
            sr = 1.0 / nr
            s2 = 1.0 / n2
            acc = jnp.zeros((L,), jnp.float32)
            for j in range(WPR):
                a1, b1 = word(st, tsel, lanes, offs, 0, j)
                ar, br = word(st, tsel, lanes, offs, 1, j)
                a2, b2 = word(st, tsel, lanes, offs, 2, j)
                acc = acc + jnp.abs(a1 * s1 + ar * sr - a2 * s2)
                acc = acc + jnp.abs(b1 * s1 + br * sr - b2 * s2)
            out_v[pl.ds(k * CHUNK + g * L, L)] = acc
            return carry

        lax.fori_loop(0, CHUNK // L, group, 0)

    pltpu.sync_copy(out_v, out.at[pl.ds(base, BPW)])


def _pack_lines(w):
    y = w.astype(jnp.bfloat16)
    z = lax.bitcast_convert_type(y.reshape(NROWS, WPR, 2), jnp.int32)
    return z.reshape(NLINES, RPL * WPR)


@jax.jit
def kernel(batch_inputs, entity_weight, relation_weight):
    bi = batch_inputs.astype(jnp.int32)
    idx = bi.reshape(NW, BPW, 3).transpose(0, 2, 1).reshape(NW * 3 * BPW)
    return _transe_sc(_pack_lines(entity_weight),
                      _pack_lines(relation_weight), idx)
